# Initial kernel scaffold; baseline (speedup 1.0000x reference)
import jax
import jax.numpy as jnp
from jax.experimental import pallas as pl


def kernel(x, edge_index, edge_feature, batch):
    raise NotImplementedError("write your pallas kernel here")


# speedup vs baseline: 96.3698x; 96.3698x over previous
import functools

import jax
import jax.numpy as jnp
from jax import lax
from jax.experimental import pallas as pl
from jax.experimental.pallas import tpu as pltpu
from jax.experimental.pallas import tpu_sc as plsc

_NC = 2
_NS = 16
_NW = _NC * _NS
_LANES = 16

_LN2 = 0.6931471805599453
_SQRT2 = 1.4142135
_LN_COEFFS = (
    -1.4097207421671691e-11,
    0.9999998711870339,
    -0.4999999005258443,
    0.33335077961154247,
    -0.25002256135353257,
    0.19936638966361775,
    -0.1655105670442295,
    0.15102536240913328,
    -0.14478062925997492,
    0.08491219291220306,
)


def _ln_approx(y):
    b = plsc.bitcast(y, jnp.int32)
    e = (b >> 23) - 127
    m = plsc.bitcast((b & 0x007FFFFF) | 0x3F800000, jnp.float32)
    big = m > _SQRT2
    m = jnp.where(big, m * 0.5, m)
    e = jnp.where(big, e + 1, e)
    u = m - 1.0
    p = jnp.full_like(u, _LN_COEFFS[-1])
    for c in _LN_COEFFS[-2::-1]:
        p = p * u + c
    return e.astype(jnp.float32) * _LN2 + p


def _sc_edge_kernel(N, E, C):
    per_w = E // _NW
    n_chunks = per_w // C
    inner_n = C // _LANES
    mesh = plsc.VectorSubcoreMesh(core_axis_name="c", subcore_axis_name="s")

    @functools.partial(
        pl.kernel,
        out_type=(
            jax.ShapeDtypeStruct((_NW * N,), jnp.float32),
            jax.ShapeDtypeStruct((_NW * _LANES,), jnp.float32),
        ),
        mesh=mesh,
        compiler_params=pltpu.CompilerParams(needs_layout_passes=False),
        scratch_types=[
            pltpu.VMEM((N,), jnp.float32),
            pltpu.VMEM((C,), jnp.int32),
            pltpu.VMEM((C,), jnp.int32),
            pltpu.VMEM((C,), jnp.float32),
        ],
    )
    def k(edges_hbm, ef_hbm, x_hbm, out_acc, out_s1, tab, srcb, dstb, efb):
        wid = lax.axis_index("s") * _NC + lax.axis_index("c")
        base = pl.multiple_of(wid * per_w, C)

        pltpu.sync_copy(x_hbm, tab)

        def chunk_a(ci, s1):
            off = pl.multiple_of(base + ci * C, C)
            pltpu.sync_copy(edges_hbm.at[pl.ds(off, C)], srcb)
            pltpu.sync_copy(edges_hbm.at[pl.ds(E + off, C)], dstb)
            pltpu.sync_copy(ef_hbm.at[pl.ds(off, C)], efb)

            def inner(j, s1):
                sl = pl.ds(j * _LANES, _LANES)
                xs = plsc.load_gather(tab, [srcb[sl]])
                xd = plsc.load_gather(tab, [dstb[sl]])
                return s1 + xs * xd * efb[sl]

            return lax.fori_loop(0, inner_n, inner, s1)

        s1 = lax.fori_loop(0, n_chunks, chunk_a, jnp.zeros((_LANES,), jnp.float32))
        efb[pl.ds(0, _LANES)] = s1
        pltpu.sync_copy(efb.at[pl.ds(0, _LANES)],
                        out_s1.at[pl.ds(wid * _LANES, _LANES)])

        def zero(j, c):
            tab[pl.ds(j * _LANES, _LANES)] = jnp.zeros((_LANES,), jnp.float32)
            return c

        lax.fori_loop(0, N // _LANES, zero, 0)

        def chunk_b(ci, c):
            off = pl.multiple_of(base + ci * C, C)
            pltpu.sync_copy(edges_hbm.at[pl.ds(E + off, C)], dstb)
            pltpu.sync_copy(ef_hbm.at[pl.ds(off, C)], efb)

            def inner(j, c2):
                sl = pl.ds(j * _LANES, _LANES)
                t = _ln_approx(1.0 - efb[sl] + 1e-6)
                plsc.addupdate_scatter(tab, [dstb[sl]], t)
                return c2

            return lax.fori_loop(0, inner_n, inner, 0)

        lax.fori_loop(0, n_chunks, chunk_b, 0)
        pltpu.sync_copy(tab, out_acc.at[pl.ds(wid * N, N)])

    return k


def _combine(acc, s1p, N):
    def body(acc_ref, s1_ref, out_ref):
        colsum = jnp.sum(acc_ref[...], axis=0)
        es = jnp.sum(jnp.exp(colsum))
        s1 = jnp.sum(s1_ref[...])
        out_ref[...] = jnp.reshape((16.0 * 200.0 * 3.0 * es - s1) / N, (1, 1))

    return pl.pallas_call(
        body,
        out_shape=jax.ShapeDtypeStruct((1, 1), jnp.float32),
    )(acc, s1p)


def kernel(x, edge_index, edge_feature, batch):
    N = x.shape[0]
    E = edge_index.shape[1]
    edges = edge_index.astype(jnp.int32).reshape(2 * E)
    ef = edge_feature.astype(jnp.float32).reshape(E)
    acc, s1p = _sc_edge_kernel(N, E, 4000)(edges, ef, x.astype(jnp.float32))
    out = _combine(acc.reshape(_NW, N), s1p.reshape(_NW, _LANES), float(N))
    return out[0, 0]


# speedup vs baseline: 101.5969x; 1.0542x over previous
import functools

import jax
import jax.numpy as jnp
from jax import lax
from jax.experimental import pallas as pl
from jax.experimental.pallas import tpu as pltpu
from jax.experimental.pallas import tpu_sc as plsc

_NC = 2
_NS = 16
_NW = _NC * _NS
_LANES = 16

_LN2 = 0.6931471805599453
_SQRT2 = 1.4142135
_LN_COEFFS = (
    -1.4097207421671691e-11,
    0.9999998711870339,
    -0.4999999005258443,
    0.33335077961154247,
    -0.25002256135353257,
    0.19936638966361775,
    -0.1655105670442295,
    0.15102536240913328,
    -0.14478062925997492,
    0.08491219291220306,
)


def _ln_approx(y):
    b = plsc.bitcast(y, jnp.int32)
    e = (b >> 23) - 127
    m = plsc.bitcast((b & 0x007FFFFF) | 0x3F800000, jnp.float32)
    big = m > _SQRT2
    m = jnp.where(big, m * 0.5, m)
    e = jnp.where(big, e + 1, e)
    u = m - 1.0
    p = jnp.full_like(u, _LN_COEFFS[-1])
    for c in _LN_COEFFS[-2::-1]:
        p = p * u + c
    return e.astype(jnp.float32) * _LN2 + p


def _sc_edge_kernel(N, E, C, U=5):
    per_w = E // _NW
    n_chunks = per_w // C
    inner_n = C // (_LANES * U)
    mesh = plsc.VectorSubcoreMesh(core_axis_name="c", subcore_axis_name="s")

    @functools.partial(
        pl.kernel,
        out_type=(
            jax.ShapeDtypeStruct((_NW * N,), jnp.float32),
            jax.ShapeDtypeStruct((_NW * _LANES,), jnp.float32),
        ),
        mesh=mesh,
        compiler_params=pltpu.CompilerParams(needs_layout_passes=False),
        scratch_types=[
            pltpu.VMEM((N,), jnp.float32),
            pltpu.VMEM((C,), jnp.int32),
            pltpu.VMEM((C,), jnp.int32),
            pltpu.VMEM((C,), jnp.float32),
        ],
    )
    def k(edges_hbm, ef_hbm, x_hbm, out_acc, out_s1, tab, srcb, dstb, efb):
        wid = lax.axis_index("s") * _NC + lax.axis_index("c")
        base = pl.multiple_of(wid * per_w, C)

        pltpu.sync_copy(x_hbm, tab)

        def chunk_a(ci, s1):
            off = pl.multiple_of(base + ci * C, C)
            pltpu.sync_copy(edges_hbm.at[pl.ds(off, C)], srcb)
            pltpu.sync_copy(edges_hbm.at[pl.ds(E + off, C)], dstb)
            pltpu.sync_copy(ef_hbm.at[pl.ds(off, C)], efb)

            def inner(j, s1):
                for u in range(U):
                    sl = pl.ds((j * U + u) * _LANES, _LANES)
                    xs = plsc.load_gather(tab, [srcb[sl]])
                    xd = plsc.load_gather(tab, [dstb[sl]])
                    s1 = s1 + xs * xd * efb[sl]
                return s1

            return lax.fori_loop(0, inner_n, inner, s1)

        s1 = lax.fori_loop(0, n_chunks, chunk_a, jnp.zeros((_LANES,), jnp.float32))
        efb[pl.ds(0, _LANES)] = s1
        pltpu.sync_copy(efb.at[pl.ds(0, _LANES)],
                        out_s1.at[pl.ds(wid * _LANES, _LANES)])

        def zero(j, c):
            for u in range(10):
                tab[pl.ds((j * 10 + u) * _LANES, _LANES)] = jnp.zeros(
                    (_LANES,), jnp.float32)
            return c

        lax.fori_loop(0, N // (_LANES * 10), zero, 0)

        def chunk_b(ci, c):
            off = pl.multiple_of(base + ci * C, C)
            pltpu.sync_copy(edges_hbm.at[pl.ds(E + off, C)], dstb)
            pltpu.sync_copy(ef_hbm.at[pl.ds(off, C)], efb)

            def inner(j, c2):
                for u in range(U):
                    sl = pl.ds((j * U + u) * _LANES, _LANES)
                    t = _ln_approx(1.0 - efb[sl] + 1e-6)
                    plsc.addupdate_scatter(tab, [dstb[sl]], t)
                return c2

            return lax.fori_loop(0, inner_n, inner, 0)

        lax.fori_loop(0, n_chunks, chunk_b, 0)
        pltpu.sync_copy(tab, out_acc.at[pl.ds(wid * N, N)])

    return k


def _combine(acc, s1p, N):
    def body(acc_ref, s1_ref, out_ref):
        colsum = jnp.sum(acc_ref[...], axis=0)
        es = jnp.sum(jnp.exp(colsum))
        s1 = jnp.sum(s1_ref[...])
        out_ref[...] = jnp.reshape((16.0 * 200.0 * 3.0 * es - s1) / N, (1, 1))

    return pl.pallas_call(
        body,
        out_shape=jax.ShapeDtypeStruct((1, 1), jnp.float32),
    )(acc, s1p)


def kernel(x, edge_index, edge_feature, batch):
    N = x.shape[0]
    E = edge_index.shape[1]
    edges = edge_index.astype(jnp.int32).reshape(2 * E)
    ef = edge_feature.astype(jnp.float32).reshape(E)
    acc, s1p = _sc_edge_kernel(N, E, 4000)(edges, ef, x.astype(jnp.float32))
    out = _combine(acc.reshape(_NW, N), s1p.reshape(_NW, _LANES), float(N))
    return out[0, 0]


# speedup vs baseline: 297.5914x; 2.9291x over previous
import functools

import jax
import jax.numpy as jnp
from jax import lax
from jax.experimental import pallas as pl
from jax.experimental.pallas import tpu as pltpu
from jax.experimental.pallas import tpu_sc as plsc

_NC = 2
_NS = 16
_NW = _NC * _NS
_LANES = 16

_LN2 = 0.6931471805599453
_SQRT2 = 1.4142135
_LN_COEFFS = (
    -1.4097207421671691e-11,
    0.9999998711870339,
    -0.4999999005258443,
    0.33335077961154247,
    -0.25002256135353257,
    0.19936638966361775,
    -0.1655105670442295,
    0.15102536240913328,
    -0.14478062925997492,
    0.08491219291220306,
)


def _ln_approx(y):
    b = plsc.bitcast(y, jnp.int32)
    e = (b >> 23) - 127
    m = plsc.bitcast((b & 0x007FFFFF) | 0x3F800000, jnp.float32)
    big = m > _SQRT2
    m = jnp.where(big, m * 0.5, m)
    e = jnp.where(big, e + 1, e)
    u = m - 1.0
    p = jnp.full_like(u, _LN_COEFFS[-1])
    for c in _LN_COEFFS[-2::-1]:
        p = p * u + c
    return e.astype(jnp.float32) * _LN2 + p


def _sc_edge_kernel(N, E, C, U=5):
    per_w = E // _NW
    n_chunks = per_w // C
    inner_n = C // (_LANES * U)
    mesh = plsc.VectorSubcoreMesh(core_axis_name="c", subcore_axis_name="s")

    @functools.partial(
        pl.kernel,
        out_type=(
            jax.ShapeDtypeStruct((_NW * N,), jnp.float32),
            jax.ShapeDtypeStruct((_NW * _LANES,), jnp.float32),
        ),
        mesh=mesh,
        compiler_params=pltpu.CompilerParams(needs_layout_passes=False),
        scratch_types=[
            pltpu.VMEM((N,), jnp.float32),
            pltpu.VMEM((C,), jnp.int32),
            pltpu.VMEM((C,), jnp.int32),
            pltpu.VMEM((C,), jnp.int32),
            pltpu.VMEM((C,), jnp.int32),
            pltpu.VMEM((C,), jnp.float32),
            pltpu.VMEM((C,), jnp.float32),
            pltpu.SemaphoreType.DMA,
            pltpu.SemaphoreType.DMA,
            pltpu.SemaphoreType.DMA,
        ],
    )
    def k(edges_hbm, ef_hbm, x_hbm, out_acc, out_s1,
          tab, srcb0, srcb1, dstb0, dstb1, efb0, efb1, semx, sem0, sem1):
        wid = lax.axis_index("s") * _NC + lax.axis_index("c")
        base = pl.multiple_of(wid * per_w, C)
        sems = (sem0, sem1)

        def start_a(ci, bufs, sem):
            off = pl.multiple_of(base + ci * C, C)
            pltpu.async_copy(edges_hbm.at[pl.ds(off, C)], bufs[0], sem)
            pltpu.async_copy(edges_hbm.at[pl.ds(E + off, C)], bufs[1], sem)
            pltpu.async_copy(ef_hbm.at[pl.ds(off, C)], bufs[2], sem)

        def wait_a(bufs, sem):
            pltpu.make_async_copy(edges_hbm.at[pl.ds(0, C)], bufs[0], sem).wait()
            pltpu.make_async_copy(edges_hbm.at[pl.ds(0, C)], bufs[1], sem).wait()
            pltpu.make_async_copy(ef_hbm.at[pl.ds(0, C)], bufs[2], sem).wait()

        def compute_a(bufs, s1):
            srcb, dstb, efb = bufs

            def inner(j, s1):
                for u in range(U):
                    sl = pl.ds((j * U + u) * _LANES, _LANES)
                    xs = plsc.load_gather(tab, [srcb[sl]])
                    xd = plsc.load_gather(tab, [dstb[sl]])
                    s1 = s1 + xs * xd * efb[sl]
                return s1

            return lax.fori_loop(0, inner_n, inner, s1)

        def start_b(ci, bufs, sem):
            off = pl.multiple_of(base + ci * C, C)
            pltpu.async_copy(edges_hbm.at[pl.ds(E + off, C)], bufs[1], sem)
            pltpu.async_copy(ef_hbm.at[pl.ds(off, C)], bufs[2], sem)

        def wait_b(bufs, sem):
            pltpu.make_async_copy(edges_hbm.at[pl.ds(0, C)], bufs[1], sem).wait()
            pltpu.make_async_copy(ef_hbm.at[pl.ds(0, C)], bufs[2], sem).wait()

        def compute_b(bufs, c):
            _, dstb, efb = bufs

            def inner(j, c2):
                idxs, ts = [], []
                for u in range(U):
                    sl = pl.ds((j * U + u) * _LANES, _LANES)
                    idxs.append(dstb[sl])
                    ts.append(_ln_approx(1.0 - efb[sl] + 1e-6))
                for idx, t in zip(idxs, ts):
                    plsc.addupdate_scatter(tab, [idx], t)
                return c2

            return lax.fori_loop(0, inner_n, inner, 0)

        bufset0 = (srcb0, dstb0, efb0)
        bufset1 = (srcb1, dstb1, efb1)

        pltpu.async_copy(x_hbm, tab, semx)
        start_a(0, bufset0, sem0)
        pltpu.make_async_copy(x_hbm, tab, semx).wait()

        def body_a(k2, s1):
            ci = 2 * k2
            start_a(ci + 1, bufset1, sem1)
            wait_a(bufset0, sem0)
            s1 = compute_a(bufset0, s1)

            @pl.when(ci + 2 < n_chunks)
            def _():
                start_a(ci + 2, bufset0, sem0)

            wait_a(bufset1, sem1)
            return compute_a(bufset1, s1)

        s1 = lax.fori_loop(0, n_chunks // 2, body_a,
                           jnp.zeros((_LANES,), jnp.float32))
        efb0[pl.ds(0, _LANES)] = s1
        pltpu.sync_copy(efb0.at[pl.ds(0, _LANES)],
                        out_s1.at[pl.ds(wid * _LANES, _LANES)])

        start_b(0, bufset0, sem0)

        def zero(j, c):
            for u in range(10):
                tab[pl.ds((j * 10 + u) * _LANES, _LANES)] = jnp.zeros(
                    (_LANES,), jnp.float32)
            return c

        lax.fori_loop(0, N // (_LANES * 10), zero, 0)

        def body_b(k2, c):
            ci = 2 * k2
            start_b(ci + 1, bufset1, sem1)
            wait_b(bufset0, sem0)
            compute_b(bufset0, 0)

            @pl.when(ci + 2 < n_chunks)
            def _():
                start_b(ci + 2, bufset0, sem0)

            wait_b(bufset1, sem1)
            compute_b(bufset1, 0)
            return c

        lax.fori_loop(0, n_chunks // 2, body_b, 0)
        pltpu.sync_copy(tab, out_acc.at[pl.ds(wid * N, N)])

    return k


def _combine(acc, s1p, N):
    def body(acc_ref, s1_ref, out_ref):
        colsum = jnp.sum(acc_ref[...], axis=0)
        es = jnp.sum(jnp.exp(colsum))
        s1 = jnp.sum(s1_ref[...])
        out_ref[...] = jnp.reshape((16.0 * 200.0 * 3.0 * es - s1) / N, (1, 1))

    return pl.pallas_call(
        body,
        out_shape=jax.ShapeDtypeStruct((1, 1), jnp.float32),
    )(acc, s1p)


def kernel(x, edge_index, edge_feature, batch):
    N = x.shape[0]
    E = edge_index.shape[1]
    edges = edge_index.astype(jnp.int32).reshape(2 * E)
    ef = edge_feature.astype(jnp.float32).reshape(E)
    acc, s1p = _sc_edge_kernel(N, E, 4000)(edges, ef, x.astype(jnp.float32))
    out = _combine(acc.reshape(_NW, N), s1p.reshape(_NW, _LANES), float(N))
    return out[0, 0]


# speedup vs baseline: 328.7497x; 1.1047x over previous
import functools

import jax
import jax.numpy as jnp
from jax import lax
from jax.experimental import pallas as pl
from jax.experimental.pallas import tpu as pltpu
from jax.experimental.pallas import tpu_sc as plsc

_NC = 2
_NS = 16
_NW = _NC * _NS
_LANES = 16

_LN2 = 0.6931471805599453
_SQRT2 = 1.4142135
_LN_COEFFS = (
    -1.4097207421671691e-11,
    0.9999998711870339,
    -0.4999999005258443,
    0.33335077961154247,
    -0.25002256135353257,
    0.19936638966361775,
    -0.1655105670442295,
    0.15102536240913328,
    -0.14478062925997492,
    0.08491219291220306,
)


def _ln_approx(y):
    b = plsc.bitcast(y, jnp.int32)
    e = (b >> 23) - 127
    m = plsc.bitcast((b & 0x007FFFFF) | 0x3F800000, jnp.float32)
    big = m > _SQRT2
    m = jnp.where(big, m * 0.5, m)
    e = jnp.where(big, e + 1, e)
    u = m - 1.0
    p = jnp.full_like(u, _LN_COEFFS[-1])
    for c in _LN_COEFFS[-2::-1]:
        p = p * u + c
    return e.astype(jnp.float32) * _LN2 + p


def _sc_edge_kernel(N, E, C=2560, U=5):
    total_chunks = E // C
    assert total_chunks * C == E
    nfull = total_chunks // _NW
    nrem = total_chunks - nfull * _NW
    assert nfull % 2 == 0
    inner_n = C // (_LANES * U)
    mesh = plsc.VectorSubcoreMesh(core_axis_name="c", subcore_axis_name="s")

    @functools.partial(
        pl.kernel,
        out_type=(
            jax.ShapeDtypeStruct((_NW * N,), jnp.float32),
            jax.ShapeDtypeStruct((_NW * _LANES,), jnp.float32),
        ),
        mesh=mesh,
        compiler_params=pltpu.CompilerParams(needs_layout_passes=False),
        scratch_types=[
            pltpu.VMEM((N,), jnp.float32),
            pltpu.VMEM((2, C), jnp.int32),
            pltpu.VMEM((2, C), jnp.int32),
            pltpu.VMEM((C,), jnp.float32),
            pltpu.VMEM((C,), jnp.float32),
            pltpu.VMEM((_LANES,), jnp.float32),
            pltpu.SemaphoreType.DMA,
            pltpu.SemaphoreType.DMA,
            pltpu.SemaphoreType.DMA,
        ],
    )
    def k(edges_hbm, ef_hbm, x_hbm, out_acc, out_s1,
          tab, eb0, eb1, efb0, efb1, s1b, semx, sem0, sem1):
        wid = lax.axis_index("s") * _NC + lax.axis_index("c")

        def off_of(k2):
            return pl.multiple_of((k2 * _NW + wid) * C, C)

        def start(k2, bufs, sem):
            off = off_of(k2)
            pltpu.async_copy(edges_hbm.at[:, pl.ds(off, C)], bufs[0], sem)
            pltpu.async_copy(ef_hbm.at[pl.ds(off, C)], bufs[1], sem)

        def wait(bufs, sem):
            pltpu.make_async_copy(edges_hbm.at[:, pl.ds(0, C)], bufs[0], sem).wait()
            pltpu.make_async_copy(ef_hbm.at[pl.ds(0, C)], bufs[1], sem).wait()

        def compute_a(bufs):
            eb, efb = bufs

            def inner(j, s1):
                for u in range(U):
                    sl = pl.ds((j * U + u) * _LANES, _LANES)
                    xs = plsc.load_gather(tab, [eb[0, sl]])
                    xd = plsc.load_gather(tab, [eb[1, sl]])
                    s1 = s1 + xs * xd * efb[sl]
                return s1

            s1 = lax.fori_loop(0, inner_n, inner,
                               jnp.zeros((_LANES,), jnp.float32))
            s1b[...] = s1b[...] + s1

        def compute_b(bufs):
            eb, efb = bufs

            def inner(j, c2):
                idxs, ts = [], []
                for u in range(U):
                    sl = pl.ds((j * U + u) * _LANES, _LANES)
                    idxs.append(eb[1, sl])
                    ts.append(_ln_approx(1.0 - efb[sl] + 1e-6))
                for idx, t in zip(idxs, ts):
                    plsc.addupdate_scatter(tab, [idx], t)
                return c2

            lax.fori_loop(0, inner_n, inner, 0)

        bufset0 = (eb0, efb0)
        bufset1 = (eb1, efb1)
        has_extra = wid < nrem

        def run_phase(compute):
            start(0, bufset0, sem0)

            def body(kp, c):
                k0 = 2 * kp
                start(k0 + 1, bufset1, sem1)
                wait(bufset0, sem0)
                compute(bufset0)

                @pl.when((k0 + 2 < nfull) | has_extra)
                def _():
                    start(k0 + 2, bufset0, sem0)

                wait(bufset1, sem1)
                compute(bufset1)
                return c

            lax.fori_loop(0, nfull // 2, body, 0)

            @pl.when(has_extra)
            def _():
                wait(bufset0, sem0)
                compute(bufset0)

        s1b[...] = jnp.zeros((_LANES,), jnp.float32)
        pltpu.async_copy(x_hbm, tab, semx)
        pltpu.make_async_copy(x_hbm, tab, semx).wait()
        run_phase(compute_a)
        efb0[pl.ds(0, _LANES)] = s1b[...]
        pltpu.sync_copy(efb0.at[pl.ds(0, _LANES)],
                        out_s1.at[pl.ds(wid * _LANES, _LANES)])

        def zero(j, c):
            for u in range(10):
                tab[pl.ds((j * 10 + u) * _LANES, _LANES)] = jnp.zeros(
                    (_LANES,), jnp.float32)
            return c

        lax.fori_loop(0, N // (_LANES * 10), zero, 0)
        run_phase(compute_b)
        pltpu.sync_copy(tab, out_acc.at[pl.ds(wid * N, N)])

    return k


def _combine(acc, s1p, N):
    def body(acc_ref, s1_ref, out_ref):
        colsum = jnp.sum(acc_ref[...], axis=0)
        es = jnp.sum(jnp.exp(colsum))
        s1 = jnp.sum(s1_ref[...])
        out_ref[...] = jnp.reshape((16.0 * 200.0 * 3.0 * es - s1) / N, (1, 1))

    return pl.pallas_call(
        body,
        out_shape=jax.ShapeDtypeStruct((1, 1), jnp.float32),
    )(acc, s1p)


def kernel(x, edge_index, edge_feature, batch):
    N = x.shape[0]
    E = edge_index.shape[1]
    edges = edge_index.astype(jnp.int32)
    ef = edge_feature.astype(jnp.float32).reshape(E)
    acc, s1p = _sc_edge_kernel(N, E)(edges, ef, x.astype(jnp.float32))
    out = _combine(acc.reshape(_NW, N), s1p.reshape(_NW, _LANES), float(N))
    return out[0, 0]


# speedup vs baseline: 358.2768x; 1.0898x over previous
import functools

import jax
import jax.numpy as jnp
from jax import lax
from jax.experimental import pallas as pl
from jax.experimental.pallas import tpu as pltpu
from jax.experimental.pallas import tpu_sc as plsc

_NC = 2
_NS = 16
_NW = _NC * _NS
_LANES = 16

_LN2 = 0.6931471805599453
_SQRT2 = 1.4142135
_LN_COEFFS = (
    -1.4097207421671691e-11,
    0.9999998711870339,
    -0.4999999005258443,
    0.33335077961154247,
    -0.25002256135353257,
    0.19936638966361775,
    -0.1655105670442295,
    0.15102536240913328,
    -0.14478062925997492,
    0.08491219291220306,
)


def _ln_approx(y):
    b = plsc.bitcast(y, jnp.int32)
    e = (b >> 23) - 127
    m = plsc.bitcast((b & 0x007FFFFF) | 0x3F800000, jnp.float32)
    big = m > _SQRT2
    m = jnp.where(big, m * 0.5, m)
    e = jnp.where(big, e + 1, e)
    u = m - 1.0
    p = jnp.full_like(u, _LN_COEFFS[-1])
    for c in _LN_COEFFS[-2::-1]:
        p = p * u + c
    return e.astype(jnp.float32) * _LN2 + p


def _sc_edge_kernel(N, Np, E, C=2560, U=5):
    total_chunks = E // C
    assert total_chunks * C == E
    nfull = total_chunks // _NW
    nrem = total_chunks - nfull * _NW
    assert nfull % 2 == 0
    inner_n = C // (_LANES * U)
    mesh = plsc.VectorSubcoreMesh(core_axis_name="c", subcore_axis_name="s")

    @functools.partial(
        pl.kernel,
        out_type=(
            jax.ShapeDtypeStruct((_NW * Np,), jnp.float32),
            jax.ShapeDtypeStruct((_NW * _LANES,), jnp.float32),
        ),
        mesh=mesh,
        compiler_params=pltpu.CompilerParams(needs_layout_passes=False),
        scratch_types=[
            pltpu.VMEM((N,), jnp.float32),
            pltpu.VMEM((2, C), jnp.int32),
            pltpu.VMEM((2, C), jnp.int32),
            pltpu.VMEM((C,), jnp.float32),
            pltpu.VMEM((C,), jnp.float32),
            pltpu.VMEM((_LANES,), jnp.float32),
            pltpu.SemaphoreType.DMA,
            pltpu.SemaphoreType.DMA,
            pltpu.SemaphoreType.DMA,
        ],
    )
    def k(edges_hbm, ef_hbm, x_hbm, out_acc, out_s1,
          tab, eb0, eb1, efb0, efb1, s1b, semx, sem0, sem1):
        wid = lax.axis_index("s") * _NC + lax.axis_index("c")

        def off_of(k2):
            return pl.multiple_of((k2 * _NW + wid) * C, C)

        def start(k2, bufs, sem):
            off = off_of(k2)
            pltpu.async_copy(edges_hbm.at[:, pl.ds(off, C)], bufs[0], sem)
            pltpu.async_copy(ef_hbm.at[pl.ds(off, C)], bufs[1], sem)

        def wait(bufs, sem):
            pltpu.make_async_copy(edges_hbm.at[:, pl.ds(0, C)], bufs[0], sem).wait()
            pltpu.make_async_copy(ef_hbm.at[pl.ds(0, C)], bufs[1], sem).wait()

        def compute_a(bufs):
            eb, efb = bufs

            def inner(j, s1):
                for u in range(U):
                    sl = pl.ds((j * U + u) * _LANES, _LANES)
                    xs = plsc.load_gather(tab, [eb[0, sl]])
                    xd = plsc.load_gather(tab, [eb[1, sl]])
                    s1 = s1 + xs * xd * efb[sl]
                return s1

            s1 = lax.fori_loop(0, inner_n, inner,
                               jnp.zeros((_LANES,), jnp.float32))
            s1b[...] = s1b[...] + s1

        def compute_b(bufs):
            eb, efb = bufs

            def inner(j, c2):
                idxs, ts = [], []
                for u in range(U):
                    sl = pl.ds((j * U + u) * _LANES, _LANES)
                    idxs.append(eb[1, sl])
                    ts.append(_ln_approx(1.0 - efb[sl] + 1e-6))
                for idx, t in zip(idxs, ts):
                    plsc.addupdate_scatter(tab, [idx], t)
                return c2

            lax.fori_loop(0, inner_n, inner, 0)

        bufset0 = (eb0, efb0)
        bufset1 = (eb1, efb1)
        has_extra = wid < nrem

        def run_phase(compute):
            start(0, bufset0, sem0)

            def body(kp, c):
                k0 = 2 * kp
                start(k0 + 1, bufset1, sem1)
                wait(bufset0, sem0)
                compute(bufset0)

                @pl.when((k0 + 2 < nfull) | has_extra)
                def _():
                    start(k0 + 2, bufset0, sem0)

                wait(bufset1, sem1)
                compute(bufset1)
                return c

            lax.fori_loop(0, nfull // 2, body, 0)

            @pl.when(has_extra)
            def _():
                wait(bufset0, sem0)
                compute(bufset0)

        s1b[...] = jnp.zeros((_LANES,), jnp.float32)
        pltpu.async_copy(x_hbm, tab, semx)
        pltpu.make_async_copy(x_hbm, tab, semx).wait()
        run_phase(compute_a)
        efb0[pl.ds(0, _LANES)] = s1b[...]
        pltpu.sync_copy(efb0.at[pl.ds(0, _LANES)],
                        out_s1.at[pl.ds(wid * _LANES, _LANES)])

        def zero(j, c):
            for u in range(10):
                tab[pl.ds((j * 10 + u) * _LANES, _LANES)] = jnp.zeros(
                    (_LANES,), jnp.float32)
            return c

        lax.fori_loop(0, N // (_LANES * 10), zero, 0)
        run_phase(compute_b)
        pltpu.sync_copy(tab, out_acc.at[pl.ds(wid * Np, N)])
        npad = Np - N
        def zpad(j, c):
            efb0[pl.ds(j * _LANES, _LANES)] = jnp.zeros((_LANES,), jnp.float32)
            return c
        lax.fori_loop(0, npad // _LANES, zpad, 0)
        pltpu.sync_copy(efb0.at[pl.ds(0, npad)],
                        out_acc.at[pl.ds(wid * Np + N, npad)])

    return k


def _combine(acc, s1p, N, Np):
    def body(acc_ref, s1_ref, out_ref):
        colsum = acc_ref[pl.ds(0, Np)]
        for w in range(1, _NW):
            colsum = colsum + acc_ref[pl.ds(w * Np, Np)]
        lane = lax.broadcasted_iota(jnp.int32, (Np,), 0)
        es = jnp.sum(jnp.where(lane < N, jnp.exp(colsum), 0.0))
        s1 = jnp.sum(s1_ref[...])
        out_ref[...] = jnp.reshape((16.0 * 200.0 * 3.0 * es - s1) / N, (1, 1))

    return pl.pallas_call(
        body,
        out_shape=jax.ShapeDtypeStruct((1, 1), jnp.float32),
    )(acc, s1p)


def kernel(x, edge_index, edge_feature, batch):
    N = x.shape[0]
    E = edge_index.shape[1]
    edges = edge_index.astype(jnp.int32)
    ef = edge_feature.astype(jnp.float32).reshape(E)
    Np = (N + 1023) // 1024 * 1024
    acc, s1p = _sc_edge_kernel(N, Np, E)(edges, ef, x.astype(jnp.float32))
    out = _combine(acc, s1p, N, Np)
    return out[0, 0]


# speedup vs baseline: 386.0448x; 1.0775x over previous
import functools

import jax
import jax.numpy as jnp
from jax import lax
from jax.experimental import pallas as pl
from jax.experimental.pallas import tpu as pltpu
from jax.experimental.pallas import tpu_sc as plsc

_NC = 2
_NS = 16
_NW = _NC * _NS
_LANES = 16

_LN2 = 0.6931471805599453
_FREXP_MAGIC = 0x3F3504F3
_LN_COEFFS = (
    6.432101468460723e-08,
    1.0000040901688685,
    -0.5000199301348585,
    0.33299597871739467,
    -0.24886378324455924,
    0.2065533459791954,
    -0.1885243878869582,
    0.11589569104592444,
)


def _ln_approx(y):
    b = plsc.bitcast(y, jnp.int32)
    t = (b - _FREXP_MAGIC) >> 23
    m = plsc.bitcast(b - (t << 23), jnp.float32)
    u = m - 1.0
    p = jnp.full_like(u, _LN_COEFFS[-1])
    for c in _LN_COEFFS[-2::-1]:
        p = p * u + c
    return t.astype(jnp.float32) * _LN2 + p


def _sc_edge_kernel(N, Np, E, C=2560, U=5):
    total_chunks = E // C
    assert total_chunks * C == E
    nfull = total_chunks // _NW
    nrem = total_chunks - nfull * _NW
    assert nfull % 2 == 0
    inner_n = C // (_LANES * U)
    mesh = plsc.VectorSubcoreMesh(core_axis_name="c", subcore_axis_name="s")

    @functools.partial(
        pl.kernel,
        out_type=(
            jax.ShapeDtypeStruct((_NW * Np,), jnp.float32),
            jax.ShapeDtypeStruct((_NW * _LANES,), jnp.float32),
        ),
        mesh=mesh,
        compiler_params=pltpu.CompilerParams(needs_layout_passes=False),
        scratch_types=[
            pltpu.VMEM((N,), jnp.float32),
            pltpu.VMEM((2, C), jnp.int32),
            pltpu.VMEM((2, C), jnp.int32),
            pltpu.VMEM((C,), jnp.float32),
            pltpu.VMEM((C,), jnp.float32),
            pltpu.VMEM((_LANES,), jnp.float32),
            pltpu.SemaphoreType.DMA,
            pltpu.SemaphoreType.DMA,
            pltpu.SemaphoreType.DMA,
        ],
    )
    def k(edges_hbm, ef_hbm, x_hbm, out_acc, out_s1,
          tab, eb0, eb1, efb0, efb1, s1b, semx, sem0, sem1):
        wid = lax.axis_index("s") * _NC + lax.axis_index("c")

        def off_of(k2):
            return pl.multiple_of((k2 * _NW + wid) * C, C)

        def start(k2, bufs, sem):
            off = off_of(k2)
            pltpu.async_copy(edges_hbm.at[:, pl.ds(off, C)], bufs[0], sem)
            pltpu.async_copy(ef_hbm.at[pl.ds(off, C)], bufs[1], sem)

        def wait(bufs, sem):
            pltpu.make_async_copy(edges_hbm.at[:, pl.ds(0, C)], bufs[0], sem).wait()
            pltpu.make_async_copy(ef_hbm.at[pl.ds(0, C)], bufs[1], sem).wait()

        def compute_a(bufs):
            eb, efb = bufs

            def inner(j, s1):
                for u in range(U):
                    sl = pl.ds((j * U + u) * _LANES, _LANES)
                    xs = plsc.load_gather(tab, [eb[0, sl]])
                    xd = plsc.load_gather(tab, [eb[1, sl]])
                    s1 = s1 + xs * xd * efb[sl]
                return s1

            s1 = lax.fori_loop(0, inner_n, inner,
                               jnp.zeros((_LANES,), jnp.float32))
            s1b[...] = s1b[...] + s1

        def compute_b(bufs):
            eb, efb = bufs

            def inner(j, c2):
                idxs, ts = [], []
                for u in range(U):
                    sl = pl.ds((j * U + u) * _LANES, _LANES)
                    idxs.append(eb[1, sl])
                    ts.append(_ln_approx(1.0 - efb[sl] + 1e-6))
                for idx, t in zip(idxs, ts):
                    plsc.addupdate_scatter(tab, [idx], t)
                return c2

            lax.fori_loop(0, inner_n, inner, 0)

        bufset0 = (eb0, efb0)
        bufset1 = (eb1, efb1)
        has_extra = wid < nrem

        def run_phase(compute):
            start(0, bufset0, sem0)

            def body(kp, c):
                k0 = 2 * kp
                start(k0 + 1, bufset1, sem1)
                wait(bufset0, sem0)
                compute(bufset0)

                @pl.when((k0 + 2 < nfull) | has_extra)
                def _():
                    start(k0 + 2, bufset0, sem0)

                wait(bufset1, sem1)
                compute(bufset1)
                return c

            lax.fori_loop(0, nfull // 2, body, 0)

            @pl.when(has_extra)
            def _():
                wait(bufset0, sem0)
                compute(bufset0)

        s1b[...] = jnp.zeros((_LANES,), jnp.float32)
        pltpu.async_copy(x_hbm, tab, semx)
        pltpu.make_async_copy(x_hbm, tab, semx).wait()
        run_phase(compute_a)
        efb0[pl.ds(0, _LANES)] = s1b[...]
        pltpu.sync_copy(efb0.at[pl.ds(0, _LANES)],
                        out_s1.at[pl.ds(wid * _LANES, _LANES)])

        def zero(j, c):
            for u in range(10):
                tab[pl.ds((j * 10 + u) * _LANES, _LANES)] = jnp.zeros(
                    (_LANES,), jnp.float32)
            return c

        lax.fori_loop(0, N // (_LANES * 10), zero, 0)
        run_phase(compute_b)
        pltpu.sync_copy(tab, out_acc.at[pl.ds(wid * Np, N)])
        npad = Np - N
        def zpad(j, c):
            efb0[pl.ds(j * _LANES, _LANES)] = jnp.zeros((_LANES,), jnp.float32)
            return c
        lax.fori_loop(0, npad // _LANES, zpad, 0)
        pltpu.sync_copy(efb0.at[pl.ds(0, npad)],
                        out_acc.at[pl.ds(wid * Np + N, npad)])

    return k


def _combine(acc, s1p, N, Np):
    def body(acc_ref, s1_ref, out_ref):
        colsum = acc_ref[pl.ds(0, Np)]
        for w in range(1, _NW):
            colsum = colsum + acc_ref[pl.ds(w * Np, Np)]
        lane = lax.broadcasted_iota(jnp.int32, (Np,), 0)
        es = jnp.sum(jnp.where(lane < N, jnp.exp(colsum), 0.0))
        s1 = jnp.sum(s1_ref[...])
        out_ref[...] = jnp.reshape((16.0 * 200.0 * 3.0 * es - s1) / N, (1, 1))

    return pl.pallas_call(
        body,
        out_shape=jax.ShapeDtypeStruct((1, 1), jnp.float32),
    )(acc, s1p)


def kernel(x, edge_index, edge_feature, batch):
    N = x.shape[0]
    E = edge_index.shape[1]
    edges = edge_index.astype(jnp.int32)
    ef = edge_feature.astype(jnp.float32).reshape(E)
    Np = (N + 1023) // 1024 * 1024
    acc, s1p = _sc_edge_kernel(N, Np, E)(edges, ef, x.astype(jnp.float32))
    out = _combine(acc, s1p, N, Np)
    return out[0, 0]


# speedup vs baseline: 408.7470x; 1.0588x over previous
import functools

import jax
import jax.numpy as jnp
from jax import lax
from jax.experimental import pallas as pl
from jax.experimental.pallas import tpu as pltpu
from jax.experimental.pallas import tpu_sc as plsc

_NC = 2
_NS = 16
_NW = _NC * _NS
_LANES = 16

_LN2 = 0.6931471805599453
_FREXP_MAGIC = 0x3F3504F3
_LN_COEFFS = (
    6.432101468460723e-08,
    1.0000040901688685,
    -0.5000199301348585,
    0.33299597871739467,
    -0.24886378324455924,
    0.2065533459791954,
    -0.1885243878869582,
    0.11589569104592444,
)


def _ln_approx(y):
    b = plsc.bitcast(y, jnp.int32)
    t = (b - _FREXP_MAGIC) >> 23
    m = plsc.bitcast(b - (t << 23), jnp.float32)
    u = m - 1.0
    p = jnp.full_like(u, _LN_COEFFS[-1])
    for c in _LN_COEFFS[-2::-1]:
        p = p * u + c
    return t.astype(jnp.float32) * _LN2 + p


def _sc_edge_kernel(N, Np, E, C=2560, U=8):
    total_chunks = E // C
    assert total_chunks * C == E
    nfull = total_chunks // _NW
    nrem = total_chunks - nfull * _NW
    assert nfull % 2 == 0
    inner_n = C // (_LANES * U)
    mesh = plsc.VectorSubcoreMesh(core_axis_name="c", subcore_axis_name="s")

    @functools.partial(
        pl.kernel,
        out_type=(
            jax.ShapeDtypeStruct((_NW * Np,), jnp.float32),
            jax.ShapeDtypeStruct((_NW * _LANES,), jnp.float32),
        ),
        mesh=mesh,
        compiler_params=pltpu.CompilerParams(needs_layout_passes=False),
        scratch_types=[
            pltpu.VMEM((N,), jnp.float32),
            pltpu.VMEM((2, C), jnp.int32),
            pltpu.VMEM((2, C), jnp.int32),
            pltpu.VMEM((C,), jnp.float32),
            pltpu.VMEM((C,), jnp.float32),
            pltpu.VMEM((_LANES,), jnp.float32),
            pltpu.SemaphoreType.DMA,
            pltpu.SemaphoreType.DMA,
            pltpu.SemaphoreType.DMA,
        ],
    )
    def k(edges_hbm, ef_hbm, x_hbm, out_acc, out_s1,
          tab, eb0, eb1, efb0, efb1, s1b, semx, sem0, sem1):
        wid = lax.axis_index("s") * _NC + lax.axis_index("c")

        def off_of(k2):
            return pl.multiple_of((k2 * _NW + wid) * C, C)

        def start(k2, bufs, sem):
            off = off_of(k2)
            pltpu.async_copy(edges_hbm.at[:, pl.ds(off, C)], bufs[0], sem)
            pltpu.async_copy(ef_hbm.at[pl.ds(off, C)], bufs[1], sem)

        def wait(bufs, sem):
            pltpu.make_async_copy(edges_hbm.at[:, pl.ds(0, C)], bufs[0], sem).wait()
            pltpu.make_async_copy(ef_hbm.at[pl.ds(0, C)], bufs[1], sem).wait()

        def compute_a(bufs):
            eb, efb = bufs

            def inner(j, s1):
                for u in range(U):
                    sl = pl.ds((j * U + u) * _LANES, _LANES)
                    xs = plsc.load_gather(tab, [eb[0, sl]])
                    xd = plsc.load_gather(tab, [eb[1, sl]])
                    s1 = s1 + xs * xd * efb[sl]
                return s1

            s1 = lax.fori_loop(0, inner_n, inner,
                               jnp.zeros((_LANES,), jnp.float32))
            s1b[...] = s1b[...] + s1

        def compute_b(bufs):
            eb, efb = bufs

            def inner(j, c2):
                idxs, ts = [], []
                for u in range(U):
                    sl = pl.ds((j * U + u) * _LANES, _LANES)
                    idxs.append(eb[1, sl])
                    ts.append(_ln_approx(1.0 - efb[sl] + 1e-6))
                for idx, t in zip(idxs, ts):
                    plsc.addupdate_scatter(tab, [idx], t)
                return c2

            lax.fori_loop(0, inner_n, inner, 0)

        bufset0 = (eb0, efb0)
        bufset1 = (eb1, efb1)
        has_extra = wid < nrem

        def run_phase(compute):
            start(0, bufset0, sem0)

            def body(kp, c):
                k0 = 2 * kp
                start(k0 + 1, bufset1, sem1)
                wait(bufset0, sem0)
                compute(bufset0)

                @pl.when((k0 + 2 < nfull) | has_extra)
                def _():
                    start(k0 + 2, bufset0, sem0)

                wait(bufset1, sem1)
                compute(bufset1)
                return c

            lax.fori_loop(0, nfull // 2, body, 0)

            @pl.when(has_extra)
            def _():
                wait(bufset0, sem0)
                compute(bufset0)

        s1b[...] = jnp.zeros((_LANES,), jnp.float32)
        pltpu.async_copy(x_hbm, tab, semx)
        pltpu.make_async_copy(x_hbm, tab, semx).wait()
        run_phase(compute_a)
        efb0[pl.ds(0, _LANES)] = s1b[...]
        pltpu.sync_copy(efb0.at[pl.ds(0, _LANES)],
                        out_s1.at[pl.ds(wid * _LANES, _LANES)])

        def zero(j, c):
            for u in range(10):
                tab[pl.ds((j * 10 + u) * _LANES, _LANES)] = jnp.zeros(
                    (_LANES,), jnp.float32)
            return c

        lax.fori_loop(0, N // (_LANES * 10), zero, 0)
        run_phase(compute_b)
        pltpu.sync_copy(tab, out_acc.at[pl.ds(wid * Np, N)])
        npad = Np - N
        def zpad(j, c):
            efb0[pl.ds(j * _LANES, _LANES)] = jnp.zeros((_LANES,), jnp.float32)
            return c
        lax.fori_loop(0, npad // _LANES, zpad, 0)
        pltpu.sync_copy(efb0.at[pl.ds(0, npad)],
                        out_acc.at[pl.ds(wid * Np + N, npad)])

    return k


def _combine(acc, s1p, N, Np):
    def body(acc_ref, s1_ref, out_ref):
        colsum = acc_ref[pl.ds(0, Np)]
        for w in range(1, _NW):
            colsum = colsum + acc_ref[pl.ds(w * Np, Np)]
        lane = lax.broadcasted_iota(jnp.int32, (Np,), 0)
        es = jnp.sum(jnp.where(lane < N, jnp.exp(colsum), 0.0))
        s1 = jnp.sum(s1_ref[...])
        out_ref[...] = jnp.reshape((16.0 * 200.0 * 3.0 * es - s1) / N, (1, 1))

    return pl.pallas_call(
        body,
        out_shape=jax.ShapeDtypeStruct((1, 1), jnp.float32),
    )(acc, s1p)


def kernel(x, edge_index, edge_feature, batch):
    N = x.shape[0]
    E = edge_index.shape[1]
    edges = edge_index.astype(jnp.int32)
    ef = edge_feature.astype(jnp.float32).reshape(E)
    Np = (N + 1023) // 1024 * 1024
    acc, s1p = _sc_edge_kernel(N, Np, E)(edges, ef, x.astype(jnp.float32))
    out = _combine(acc, s1p, N, Np)
    return out[0, 0]
